# Initial kernel scaffold; baseline (speedup 1.0000x reference)
import jax
import jax.numpy as jnp
from jax.experimental import pallas as pl


def kernel(nids, edge_index1, weights1, edge_index2, weights2, pos_edges, neg_edges, emb_table, bias, Q1w, Q1b, W1w, W1b, Q2w, Q2b, W2w, W2b):
    raise NotImplementedError("write your pallas kernel here")


# speedup vs baseline: 1.0508x; 1.0508x over previous
import functools

import jax
import jax.numpy as jnp
from jax.experimental import pallas as pl
from jax.experimental.pallas import tpu as pltpu

N = 10000
D = 128
RB = 1000


def _mm_relu_body(h_ref, w_ref, b_ref, o_ref):
    acc = jnp.dot(h_ref[...], w_ref[...], preferred_element_type=jnp.float32)
    o_ref[...] = jax.nn.relu(acc + b_ref[...])


def _mm_relu(h, w, b):
    n = h.shape[0]
    return pl.pallas_call(
        _mm_relu_body,
        grid=(n // RB,),
        in_specs=[
            pl.BlockSpec((RB, D), lambda i: (i, 0)),
            pl.BlockSpec((D, D), lambda i: (0, 0)),
            pl.BlockSpec((1, D), lambda i: (0, 0)),
        ],
        out_specs=pl.BlockSpec((RB, D), lambda i: (i, 0)),
        out_shape=jax.ShapeDtypeStruct((n, D), jnp.float32),
    )(h, w, b.reshape(1, D))


def _combine_body(n_ref, ws_ref, h_ref, wt_ref, wb_ref, b_ref, add_ref, o_ref):
    x = n_ref[...] / ws_ref[...]
    z = jnp.dot(x, wt_ref[...], preferred_element_type=jnp.float32)
    z = z + jnp.dot(h_ref[...], wb_ref[...], preferred_element_type=jnp.float32)
    z = jax.nn.relu(z + b_ref[...])
    zn = jnp.sqrt(jnp.sum(z * z, axis=1, keepdims=True))
    zn = jnp.where(zn == 0.0, 1.0, zn)
    o_ref[...] = z / zn + add_ref[...]


def _combine(nagg, ws, h, Ww, Wb, add):
    n = h.shape[0]
    return pl.pallas_call(
        _combine_body,
        grid=(n // RB,),
        in_specs=[
            pl.BlockSpec((RB, D), lambda i: (i, 0)),
            pl.BlockSpec((RB, 1), lambda i: (i, 0)),
            pl.BlockSpec((RB, D), lambda i: (i, 0)),
            pl.BlockSpec((D, D), lambda i: (0, 0)),
            pl.BlockSpec((D, D), lambda i: (0, 0)),
            pl.BlockSpec((1, D), lambda i: (0, 0)),
            pl.BlockSpec((RB, D), lambda i: (i, 0)),
        ],
        out_specs=pl.BlockSpec((RB, D), lambda i: (i, 0)),
        out_shape=jax.ShapeDtypeStruct((n, D), jnp.float32),
    )(nagg, ws, h, Ww[:D], Ww[D:], Wb.reshape(1, D), add)


def _sage_layer(h, src, dst, w, Qw, Qb, Ww, Wb, add):
    n_src = _mm_relu(h, Qw, Qb)
    m = n_src[src] * w[:, None]
    nagg = jax.ops.segment_sum(m, dst, num_segments=N)
    ws = jnp.clip(jax.ops.segment_sum(w, dst, num_segments=N), 1.0, None)[:, None]
    return _combine(nagg, ws, h, Ww, Wb, add)


def kernel(nids, edge_index1, weights1, edge_index2, weights2, pos_edges, neg_edges,
           emb_table, bias, Q1w, Q1b, W1w, W1b, Q2w, Q2b, W2w, W2b):
    h0 = jnp.take(emb_table, nids, axis=0)
    zero = jnp.zeros((N, D), jnp.float32)
    h1 = _sage_layer(h0, edge_index1[0], edge_index1[1], weights1, Q1w, Q1b, W1w, W1b, zero)
    h_item = _sage_layer(h1, edge_index2[0], edge_index2[1], weights2, Q2w, Q2b, W2w, W2b, h0)

    bn = bias[nids, 0]

    def _score(edges):
        s = jnp.sum(h_item[edges[0]] * h_item[edges[1]], axis=1, keepdims=True)
        return s + bn[edges[0], None] + bn[edges[1], None]

    return jnp.concatenate([_score(pos_edges), _score(neg_edges)], axis=0)


# speedup vs baseline: 4.4773x; 4.2610x over previous
import functools

import jax
import jax.numpy as jnp
from jax import lax
from jax.experimental import pallas as pl
from jax.experimental.pallas import tpu as pltpu
from jax.experimental.pallas import tpu_sc as plsc

N = 10000
D = 128
E = 320000
RB = 1000
NT = 32
NSUB = 16
EPT = E // NT
CH = 80
NCH = EPT // CH
SG = 5
NGRP = NCH // SG
WSW = 16
NP = 10240
RPT = NP // NSUB
ZR = 128


def _mm_relu_body(h_ref, w_ref, b_ref, o_ref):
    acc = jnp.dot(h_ref[...], w_ref[...], preferred_element_type=jnp.float32)
    o_ref[...] = jax.nn.relu(acc + b_ref[...])


def _mm_relu(h, w, b):
    n = h.shape[0]
    return pl.pallas_call(
        _mm_relu_body,
        grid=(n // RB,),
        in_specs=[
            pl.BlockSpec((RB, D), lambda i: (i, 0)),
            pl.BlockSpec((D, D), lambda i: (0, 0)),
            pl.BlockSpec((1, D), lambda i: (0, 0)),
        ],
        out_specs=pl.BlockSpec((RB, D), lambda i: (i, 0)),
        out_shape=jax.ShapeDtypeStruct((n, D), jnp.float32),
    )(h, w, b.reshape(1, D))


def _combine_body(a0_ref, a1_ref, w0_ref, w1_ref, h_ref, wt_ref, wb_ref,
                  b_ref, add_ref, o_ref):
    nagg = a0_ref[...] + a1_ref[...]
    ws = jnp.sum(w0_ref[...] + w1_ref[...], axis=1, keepdims=True)
    ws = jnp.clip(ws, 1.0, None)
    z = jnp.dot(nagg / ws, wt_ref[...], preferred_element_type=jnp.float32)
    z = z + jnp.dot(h_ref[...], wb_ref[...], preferred_element_type=jnp.float32)
    z = jax.nn.relu(z + b_ref[...])
    zn = jnp.sqrt(jnp.sum(z * z, axis=1, keepdims=True))
    zn = jnp.where(zn == 0.0, 1.0, zn)
    o_ref[...] = z / zn + add_ref[...]


def _combine(a0, a1, w0, w1, h, Ww, Wb, add):
    nb = N // RB
    return pl.pallas_call(
        _combine_body,
        grid=(nb,),
        in_specs=[
            pl.BlockSpec((RB, D), lambda i: (i, 0)),
            pl.BlockSpec((RB, D), lambda i: (i, 0)),
            pl.BlockSpec((RB, WSW), lambda i: (i, 0)),
            pl.BlockSpec((RB, WSW), lambda i: (i, 0)),
            pl.BlockSpec((RB, D), lambda i: (i, 0)),
            pl.BlockSpec((D, D), lambda i: (0, 0)),
            pl.BlockSpec((D, D), lambda i: (0, 0)),
            pl.BlockSpec((1, D), lambda i: (0, 0)),
            pl.BlockSpec((RB, D), lambda i: (i, 0)),
        ],
        out_specs=pl.BlockSpec((RB, D), lambda i: (i, 0)),
        out_shape=jax.ShapeDtypeStruct((N, D), jnp.float32),
    )(a0, a1, w0, w1, h, Ww[:D], Ww[D:], Wb.reshape(1, D), add)


def _splat_lane(v, e):
    idx = jnp.full((16, 1), e, jnp.int32)
    return lax.gather(
        v, idx,
        dimension_numbers=lax.GatherDimensionNumbers(
            offset_dims=(), collapsed_slice_dims=(0,), start_index_map=(0,)),
        slice_sizes=(1,),
        mode=lax.GatherScatterMode.PROMISE_IN_BOUNDS)

def _edge_body(nsrc_hbm, src_hbm, dst_hbm, w_hbm, acc_out, ws_out,
               src_v, dst_v, w_v, rows_v, wblk_v,
               acc_sh, ws_sh, sem):
    c = lax.axis_index("c")
    s = lax.axis_index("s")
    t = c * NSUB + s
    zero16 = jnp.zeros((16,), jnp.float32)
    lane_iota = lax.iota(jnp.int32, 16)

    def _zrow(i, _):
        for dd in range(D // 16):
            rows_v[i, pl.ds(dd * 16, 16)] = zero16
        wblk_v[i, :] = zero16
        return 0
    lax.fori_loop(0, CH, _zrow, 0)

    for k in range(RPT // CH):
        pltpu.sync_copy(rows_v, acc_sh.at[pl.ds(s * RPT + k * CH, CH)])
        pltpu.sync_copy(wblk_v, ws_sh.at[pl.ds(s * RPT + k * CH, CH)])

    plsc.subcore_barrier()

    def _group(g, _):
        pltpu.sync_copy(src_hbm.at[t * NGRP + g], src_v)
        pltpu.sync_copy(dst_hbm.at[t * NGRP + g], dst_v)
        pltpu.sync_copy(w_hbm.at[t * NGRP + g], w_v)

        def _chunk(j, _):
            pltpu.async_copy(nsrc_hbm.at[src_v.at[j]], rows_v, sem).wait()
            for gg in range(CH // 16):
                wv = w_v[j, pl.ds(gg * 16, 16)]
                for e in range(16):
                    r = gg * 16 + e
                    wsp = _splat_lane(wv, e)
                    wblk_v[r, :] = jnp.where(lane_iota == 0, wsp, 0.0)
                    for dd in range(D // 16):
                        sl = pl.ds(dd * 16, 16)
                        rows_v[r, sl] = rows_v[r, sl] * wsp
            pltpu.sync_copy(rows_v, acc_sh.at[dst_v.at[j]], add=True)
            pltpu.sync_copy(wblk_v, ws_sh.at[dst_v.at[j]], add=True)
            return 0

        lax.fori_loop(0, SG, _chunk, 0)
        return 0

    lax.fori_loop(0, NGRP, _group, 0)

    plsc.subcore_barrier()

    for k in range(RPT // CH):
        pltpu.sync_copy(acc_sh.at[pl.ds(s * RPT + k * CH, CH)], rows_v)
        pltpu.sync_copy(rows_v, acc_out.at[pl.ds(c * NP + s * RPT + k * CH, CH)])
        pltpu.sync_copy(ws_sh.at[pl.ds(s * RPT + k * CH, CH)], wblk_v)
        pltpu.sync_copy(wblk_v, ws_out.at[pl.ds(c * NP + s * RPT + k * CH, CH)])


@jax.jit
def _edge_agg(n_src, src_r, dst_r, w_r):
    mesh = plsc.VectorSubcoreMesh(core_axis_name="c", subcore_axis_name="s")
    f = pl.kernel(
        _edge_body,
        out_type=(
            jax.ShapeDtypeStruct((2 * NP, D), jnp.float32),
            jax.ShapeDtypeStruct((2 * NP, WSW), jnp.float32),
        ),
        mesh=mesh,
        scratch_types=[
            pltpu.VMEM((SG, CH), jnp.int32),
            pltpu.VMEM((SG, CH), jnp.int32),
            pltpu.VMEM((SG, CH), jnp.float32),
            pltpu.VMEM((CH, D), jnp.float32),
            pltpu.VMEM((CH, WSW), jnp.float32),
            pltpu.VMEM_SHARED((NP, D), jnp.float32),
            pltpu.VMEM_SHARED((NP, WSW), jnp.float32),
            pltpu.SemaphoreType.DMA,
        ],
        compiler_params=pltpu.CompilerParams(use_tc_tiling_on_sc=False),
    )
    return f(n_src, src_r, dst_r, w_r)


def _sage_layer(h, src_r, dst_r, w_r, Qw, Qb, Ww, Wb, add):
    n_src = _mm_relu(h, Qw, Qb)
    acc2, ws2 = _edge_agg(n_src, src_r, dst_r, w_r)
    a = acc2.reshape(2, NP, D)
    w = ws2.reshape(2, NP, WSW)
    return _combine(a[0], a[1], w[0], w[1], h, Ww, Wb, add)


def kernel(nids, edge_index1, weights1, edge_index2, weights2, pos_edges, neg_edges,
           emb_table, bias, Q1w, Q1b, W1w, W1b, Q2w, Q2b, W2w, W2b):
    h0 = jnp.take(emb_table, nids, axis=0)
    zero = jnp.zeros((N, D), jnp.float32)
    esh = (NT * NGRP, SG, CH)
    s1, d1 = edge_index1[0].reshape(esh), edge_index1[1].reshape(esh)
    s2, d2 = edge_index2[0].reshape(esh), edge_index2[1].reshape(esh)
    w1r = weights1.reshape(esh)
    w2r = weights2.reshape(esh)
    h1 = _sage_layer(h0, s1, d1, w1r, Q1w, Q1b, W1w, W1b, zero)
    h_item = _sage_layer(h1, s2, d2, w2r, Q2w, Q2b, W2w, W2b, h0)

    bn = bias[nids, 0]

    def _score(edges):
        s = jnp.sum(h_item[edges[0]] * h_item[edges[1]], axis=1, keepdims=True)
        return s + bn[edges[0], None] + bn[edges[1], None]

    return jnp.concatenate([_score(pos_edges), _score(neg_edges)], axis=0)


# speedup vs baseline: 6.7612x; 1.5101x over previous
import functools

import jax
import jax.numpy as jnp
from jax import lax
from jax.experimental import pallas as pl
from jax.experimental.pallas import tpu as pltpu
from jax.experimental.pallas import tpu_sc as plsc

N = 10000
D = 128
E = 320000
RB = 1000
NT = 32
NSUB = 16
EPT = E // NT
CH = 80
NCH = EPT // CH
SG = 25
NGRP = NCH // SG
WSW = 16
NP = 10240
RPT = NP // NSUB
ZR = 128


def _mm_relu_body(h_ref, w_ref, b_ref, o_ref):
    acc = jnp.dot(h_ref[...], w_ref[...], preferred_element_type=jnp.float32)
    o_ref[...] = jax.nn.relu(acc + b_ref[...])


def _mm_relu(h, w, b):
    n = h.shape[0]
    return pl.pallas_call(
        _mm_relu_body,
        grid=(n // RB,),
        in_specs=[
            pl.BlockSpec((RB, D), lambda i: (i, 0)),
            pl.BlockSpec((D, D), lambda i: (0, 0)),
            pl.BlockSpec((1, D), lambda i: (0, 0)),
        ],
        out_specs=pl.BlockSpec((RB, D), lambda i: (i, 0)),
        out_shape=jax.ShapeDtypeStruct((n, D), jnp.float32),
    )(h, w, b.reshape(1, D))


def _combine_body(a0_ref, a1_ref, w0_ref, w1_ref, h_ref, wt_ref, wb_ref,
                  b_ref, add_ref, o_ref):
    nagg = a0_ref[...] + a1_ref[...]
    ws = jnp.sum(w0_ref[...] + w1_ref[...], axis=1, keepdims=True)
    ws = jnp.clip(ws, 1.0, None)
    z = jnp.dot(nagg / ws, wt_ref[...], preferred_element_type=jnp.float32)
    z = z + jnp.dot(h_ref[...], wb_ref[...], preferred_element_type=jnp.float32)
    z = jax.nn.relu(z + b_ref[...])
    zn = jnp.sqrt(jnp.sum(z * z, axis=1, keepdims=True))
    zn = jnp.where(zn == 0.0, 1.0, zn)
    o_ref[...] = z / zn + add_ref[...]


def _combine(a0, a1, w0, w1, h, Ww, Wb, add):
    nb = N // RB
    return pl.pallas_call(
        _combine_body,
        grid=(nb,),
        in_specs=[
            pl.BlockSpec((RB, D), lambda i: (i, 0)),
            pl.BlockSpec((RB, D), lambda i: (i, 0)),
            pl.BlockSpec((RB, WSW), lambda i: (i, 0)),
            pl.BlockSpec((RB, WSW), lambda i: (i, 0)),
            pl.BlockSpec((RB, D), lambda i: (i, 0)),
            pl.BlockSpec((D, D), lambda i: (0, 0)),
            pl.BlockSpec((D, D), lambda i: (0, 0)),
            pl.BlockSpec((1, D), lambda i: (0, 0)),
            pl.BlockSpec((RB, D), lambda i: (i, 0)),
        ],
        out_specs=pl.BlockSpec((RB, D), lambda i: (i, 0)),
        out_shape=jax.ShapeDtypeStruct((N, D), jnp.float32),
    )(a0, a1, w0, w1, h, Ww[:D], Ww[D:], Wb.reshape(1, D), add)


def _splat_lane(v, e):
    idx = jnp.full((16, 1), e, jnp.int32)
    return lax.gather(
        v, idx,
        dimension_numbers=lax.GatherDimensionNumbers(
            offset_dims=(), collapsed_slice_dims=(0,), start_index_map=(0,)),
        slice_sizes=(1,),
        mode=lax.GatherScatterMode.PROMISE_IN_BOUNDS)

def _edge_body(nsrc_hbm, src_hbm, dst_hbm, w_hbm, acc_out, ws_out,
               src_v, dst_v, w_v, rows0, rows1, rows2, wblk_v,
               acc_sh, ws_sh, gsem, ssem):
    c = lax.axis_index("c")
    s = lax.axis_index("s")
    t = c * NSUB + s
    zero16 = jnp.zeros((16,), jnp.float32)
    lane_iota = lax.iota(jnp.int32, 16)
    rowsb = [rows0, rows1, rows2]

    def _zrow(i, _):
        for dd in range(D // 16):
            rows0[i, pl.ds(dd * 16, 16)] = zero16
        wblk_v[i, :] = zero16
        return 0
    lax.fori_loop(0, CH, _zrow, 0)

    for k in range(RPT // CH):
        pltpu.sync_copy(rows0, acc_sh.at[pl.ds(s * RPT + k * CH, CH)])
        pltpu.sync_copy(wblk_v, ws_sh.at[pl.ds(s * RPT + k * CH, CH)])

    plsc.subcore_barrier()

    def _scale(rows, cc):
        def _sixteen(g, _):
            wv = w_v[cc, pl.ds(g * 16, 16)]
            for e in range(16):
                r = g * 16 + e
                wsp = _splat_lane(wv, e)
                wblk_v[r, :] = jnp.where(lane_iota == 0, wsp, 0.0)
                for dd in range(D // 16):
                    sl = pl.ds(dd * 16, 16)
                    rows[r, sl] = rows[r, sl] * wsp
            return 0
        lax.fori_loop(0, CH // 16, _sixteen, 0)

    def _substep(cc, b, bn, issue_next, wait_ssem):
        rows = rowsb[b]
        pltpu.make_async_copy(nsrc_hbm.at[src_v.at[cc]], rows, gsem.at[b]).wait()
        _scale(rows, cc)
        pltpu.sync_copy(wblk_v, ws_sh.at[dst_v.at[cc]], add=True)
        if issue_next:
            if wait_ssem:
                pltpu.make_async_copy(
                    rowsb[bn], acc_sh.at[dst_v.at[0]], ssem.at[bn]).wait()
            pltpu.async_copy(nsrc_hbm.at[src_v.at[cc + 2]], rowsb[bn],
                             gsem.at[bn])
        pltpu.async_copy(rows, acc_sh.at[dst_v.at[cc]], ssem.at[b], add=True)

    def _group(g, _):
        base = t * NGRP + g
        pltpu.sync_copy(src_hbm.at[base], src_v)
        pltpu.sync_copy(dst_hbm.at[base], dst_v)
        pltpu.sync_copy(w_hbm.at[base], w_v)
        pltpu.async_copy(nsrc_hbm.at[src_v.at[0]], rowsb[0], gsem.at[0])
        pltpu.async_copy(nsrc_hbm.at[src_v.at[1]], rowsb[1], gsem.at[1])
        _substep(0, 0, 2, True, False)

        def _triple(jj, _):
            cb = 3 * jj
            _substep(cb + 1, 1, 0, True, True)
            _substep(cb + 2, 2, 1, True, True)
            _substep(cb + 3, 0, 2, True, True)
            return 0
        lax.fori_loop(0, 7, _triple, 0)

        _substep(SG - 3, 1, 0, True, True)
        _substep(SG - 2, 2, 1, False, False)
        _substep(SG - 1, 0, 2, False, False)
        for b in range(3):
            pltpu.make_async_copy(
                rowsb[b], acc_sh.at[dst_v.at[0]], ssem.at[b]).wait()
        return 0

    lax.fori_loop(0, NGRP, _group, 0)

    plsc.subcore_barrier()

    for k in range(RPT // CH):
        pltpu.sync_copy(acc_sh.at[pl.ds(s * RPT + k * CH, CH)], rows0)
        pltpu.sync_copy(rows0, acc_out.at[pl.ds(c * NP + s * RPT + k * CH, CH)])
        pltpu.sync_copy(ws_sh.at[pl.ds(s * RPT + k * CH, CH)], wblk_v)
        pltpu.sync_copy(wblk_v, ws_out.at[pl.ds(c * NP + s * RPT + k * CH, CH)])


@jax.jit
def _edge_agg(n_src, src_r, dst_r, w_r):
    mesh = plsc.VectorSubcoreMesh(core_axis_name="c", subcore_axis_name="s")
    f = pl.kernel(
        _edge_body,
        out_type=(
            jax.ShapeDtypeStruct((2 * NP, D), jnp.float32),
            jax.ShapeDtypeStruct((2 * NP, WSW), jnp.float32),
        ),
        mesh=mesh,
        scratch_types=[
            pltpu.VMEM((SG, CH), jnp.int32),
            pltpu.VMEM((SG, CH), jnp.int32),
            pltpu.VMEM((SG, CH), jnp.float32),
            pltpu.VMEM((CH, D), jnp.float32),
            pltpu.VMEM((CH, D), jnp.float32),
            pltpu.VMEM((CH, D), jnp.float32),
            pltpu.VMEM((CH, WSW), jnp.float32),
            pltpu.VMEM_SHARED((NP, D), jnp.float32),
            pltpu.VMEM_SHARED((NP, WSW), jnp.float32),
            pltpu.SemaphoreType.DMA((3,)),
            pltpu.SemaphoreType.DMA((3,)),
        ],
        compiler_params=pltpu.CompilerParams(use_tc_tiling_on_sc=False),
    )
    return f(n_src, src_r, dst_r, w_r)


def _sage_layer(h, src_r, dst_r, w_r, Qw, Qb, Ww, Wb, add):
    n_src = _mm_relu(h, Qw, Qb)
    acc2, ws2 = _edge_agg(n_src, src_r, dst_r, w_r)
    a = acc2.reshape(2, NP, D)
    w = ws2.reshape(2, NP, WSW)
    return _combine(a[0], a[1], w[0], w[1], h, Ww, Wb, add)


def kernel(nids, edge_index1, weights1, edge_index2, weights2, pos_edges, neg_edges,
           emb_table, bias, Q1w, Q1b, W1w, W1b, Q2w, Q2b, W2w, W2b):
    h0 = jnp.take(emb_table, nids, axis=0)
    zero = jnp.zeros((N, D), jnp.float32)
    esh = (NT * NGRP, SG, CH)
    s1, d1 = edge_index1[0].reshape(esh), edge_index1[1].reshape(esh)
    s2, d2 = edge_index2[0].reshape(esh), edge_index2[1].reshape(esh)
    w1r = weights1.reshape(esh)
    w2r = weights2.reshape(esh)
    h1 = _sage_layer(h0, s1, d1, w1r, Q1w, Q1b, W1w, W1b, zero)
    h_item = _sage_layer(h1, s2, d2, w2r, Q2w, Q2b, W2w, W2b, h0)

    bn = bias[nids, 0]

    def _score(edges):
        s = jnp.sum(h_item[edges[0]] * h_item[edges[1]], axis=1, keepdims=True)
        return s + bn[edges[0], None] + bn[edges[1], None]

    return jnp.concatenate([_score(pos_edges), _score(neg_edges)], axis=0)


# speedup vs baseline: 6.8693x; 1.0160x over previous
import functools

import jax
import jax.numpy as jnp
from jax import lax
from jax.experimental import pallas as pl
from jax.experimental.pallas import tpu as pltpu
from jax.experimental.pallas import tpu_sc as plsc

N = 10000
D = 128
E = 320000
RB = 1000
NT = 32
NSUB = 16
EPT = E // NT
CH = 80
NCH = EPT // CH
SG = 25
NGRP = NCH // SG
WSW = 16
NP = 10240
RPT = NP // NSUB
ZR = 128


def _mm_relu_body(h_ref, w_ref, b_ref, o_ref):
    acc = jnp.dot(h_ref[...], w_ref[...], preferred_element_type=jnp.float32)
    o_ref[...] = jax.nn.relu(acc + b_ref[...])


def _mm_relu(h, w, b):
    n = h.shape[0]
    return pl.pallas_call(
        _mm_relu_body,
        grid=(n // RB,),
        in_specs=[
            pl.BlockSpec((RB, D), lambda i: (i, 0)),
            pl.BlockSpec((D, D), lambda i: (0, 0)),
            pl.BlockSpec((1, D), lambda i: (0, 0)),
        ],
        out_specs=pl.BlockSpec((RB, D), lambda i: (i, 0)),
        out_shape=jax.ShapeDtypeStruct((n, D), jnp.float32),
    )(h, w, b.reshape(1, D))


def _combine_body(a0_ref, a1_ref, w0_ref, w1_ref, h_ref, wt_ref, wb_ref,
                  b_ref, add_ref, o_ref):
    nagg = a0_ref[...] + a1_ref[...]
    ws = jnp.sum(w0_ref[...] + w1_ref[...], axis=1, keepdims=True)
    ws = jnp.clip(ws, 1.0, None)
    z = jnp.dot(nagg / ws, wt_ref[...], preferred_element_type=jnp.float32)
    z = z + jnp.dot(h_ref[...], wb_ref[...], preferred_element_type=jnp.float32)
    z = jax.nn.relu(z + b_ref[...])
    zn = jnp.sqrt(jnp.sum(z * z, axis=1, keepdims=True))
    zn = jnp.where(zn == 0.0, 1.0, zn)
    o_ref[...] = z / zn + add_ref[...]


def _combine(a0, a1, w0, w1, h, Ww, Wb, add):
    nb = N // RB
    return pl.pallas_call(
        _combine_body,
        grid=(nb,),
        in_specs=[
            pl.BlockSpec((RB, D), lambda i: (i, 0)),
            pl.BlockSpec((RB, D), lambda i: (i, 0)),
            pl.BlockSpec((RB, WSW), lambda i: (i, 0)),
            pl.BlockSpec((RB, WSW), lambda i: (i, 0)),
            pl.BlockSpec((RB, D), lambda i: (i, 0)),
            pl.BlockSpec((D, D), lambda i: (0, 0)),
            pl.BlockSpec((D, D), lambda i: (0, 0)),
            pl.BlockSpec((1, D), lambda i: (0, 0)),
            pl.BlockSpec((RB, D), lambda i: (i, 0)),
        ],
        out_specs=pl.BlockSpec((RB, D), lambda i: (i, 0)),
        out_shape=jax.ShapeDtypeStruct((N, D), jnp.float32),
    )(a0, a1, w0, w1, h, Ww[:D], Ww[D:], Wb.reshape(1, D), add)


def _splat_lane(v, e):
    idx = jnp.full((16, 1), e, jnp.int32)
    return lax.gather(
        v, idx,
        dimension_numbers=lax.GatherDimensionNumbers(
            offset_dims=(), collapsed_slice_dims=(0,), start_index_map=(0,)),
        slice_sizes=(1,),
        mode=lax.GatherScatterMode.PROMISE_IN_BOUNDS)

def _edge_body(nsrc_hbm, src_hbm, dst_hbm, w_hbm, acc_out, ws_out,
               src_v, dst_v, w_v, rows0, rows1, rows2, wblk_v,
               acc_sh, ws_sh, gsem, ssem, wsem):
    c = lax.axis_index("c")
    s = lax.axis_index("s")
    t = c * NSUB + s
    zero16 = jnp.zeros((16,), jnp.float32)
    lane_iota = lax.iota(jnp.int32, 16)
    rowsb = [rows0, rows1, rows2]

    def _zrow(i, _):
        for dd in range(D // 16):
            rows0[i, pl.ds(dd * 16, 16)] = zero16
        wblk_v[i, :] = zero16
        return 0
    lax.fori_loop(0, CH, _zrow, 0)

    for k in range(RPT // CH):
        pltpu.sync_copy(rows0, acc_sh.at[pl.ds(s * RPT + k * CH, CH)])
        pltpu.sync_copy(wblk_v, ws_sh.at[pl.ds(s * RPT + k * CH, CH)])

    plsc.subcore_barrier()

    def _scale(rows, cc):
        def _sixteen(g, _):
            wv = w_v[cc, pl.ds(g * 16, 16)]
            for e in range(16):
                r = g * 16 + e
                wsp = _splat_lane(wv, e)
                wblk_v[r, :] = jnp.where(lane_iota == 0, wsp, 0.0)
                for dd in range(D // 16):
                    sl = pl.ds(dd * 16, 16)
                    rows[r, sl] = rows[r, sl] * wsp
            return 0
        lax.fori_loop(0, CH // 16, _sixteen, 0)

    def _substep(cc, b, bn, issue_next, wait_ssem, wait_wsem=True):
        rows = rowsb[b]
        pltpu.make_async_copy(nsrc_hbm.at[src_v.at[cc]], rows, gsem.at[b]).wait()
        if wait_wsem:
            pltpu.make_async_copy(wblk_v, ws_sh.at[dst_v.at[0]], wsem).wait()
        _scale(rows, cc)
        pltpu.async_copy(wblk_v, ws_sh.at[dst_v.at[cc]], wsem, add=True)
        if issue_next:
            if wait_ssem:
                pltpu.make_async_copy(
                    rowsb[bn], acc_sh.at[dst_v.at[0]], ssem.at[bn]).wait()
            pltpu.async_copy(nsrc_hbm.at[src_v.at[cc + 2]], rowsb[bn],
                             gsem.at[bn])
        pltpu.async_copy(rows, acc_sh.at[dst_v.at[cc]], ssem.at[b], add=True)

    def _group(g, _):
        base = t * NGRP + g
        pltpu.sync_copy(src_hbm.at[base], src_v)
        pltpu.sync_copy(dst_hbm.at[base], dst_v)
        pltpu.sync_copy(w_hbm.at[base], w_v)
        pltpu.async_copy(nsrc_hbm.at[src_v.at[0]], rowsb[0], gsem.at[0])
        pltpu.async_copy(nsrc_hbm.at[src_v.at[1]], rowsb[1], gsem.at[1])
        _substep(0, 0, 2, True, False, wait_wsem=False)

        def _triple(jj, _):
            cb = 3 * jj
            _substep(cb + 1, 1, 0, True, True)
            _substep(cb + 2, 2, 1, True, True)
            _substep(cb + 3, 0, 2, True, True)
            return 0
        lax.fori_loop(0, 7, _triple, 0)

        _substep(SG - 3, 1, 0, True, True)
        _substep(SG - 2, 2, 1, False, False)
        _substep(SG - 1, 0, 2, False, False)
        for b in range(3):
            pltpu.make_async_copy(
                rowsb[b], acc_sh.at[dst_v.at[0]], ssem.at[b]).wait()
        pltpu.make_async_copy(wblk_v, ws_sh.at[dst_v.at[0]], wsem).wait()
        return 0

    lax.fori_loop(0, NGRP, _group, 0)

    plsc.subcore_barrier()

    for k in range(RPT // CH):
        pltpu.sync_copy(acc_sh.at[pl.ds(s * RPT + k * CH, CH)], rows0)
        pltpu.sync_copy(rows0, acc_out.at[pl.ds(c * NP + s * RPT + k * CH, CH)])
        pltpu.sync_copy(ws_sh.at[pl.ds(s * RPT + k * CH, CH)], wblk_v)
        pltpu.sync_copy(wblk_v, ws_out.at[pl.ds(c * NP + s * RPT + k * CH, CH)])


@jax.jit
def _edge_agg(n_src, src_r, dst_r, w_r):
    mesh = plsc.VectorSubcoreMesh(core_axis_name="c", subcore_axis_name="s")
    f = pl.kernel(
        _edge_body,
        out_type=(
            jax.ShapeDtypeStruct((2 * NP, D), jnp.float32),
            jax.ShapeDtypeStruct((2 * NP, WSW), jnp.float32),
        ),
        mesh=mesh,
        scratch_types=[
            pltpu.VMEM((SG, CH), jnp.int32),
            pltpu.VMEM((SG, CH), jnp.int32),
            pltpu.VMEM((SG, CH), jnp.float32),
            pltpu.VMEM((CH, D), jnp.float32),
            pltpu.VMEM((CH, D), jnp.float32),
            pltpu.VMEM((CH, D), jnp.float32),
            pltpu.VMEM((CH, WSW), jnp.float32),
            pltpu.VMEM_SHARED((NP, D), jnp.float32),
            pltpu.VMEM_SHARED((NP, WSW), jnp.float32),
            pltpu.SemaphoreType.DMA((3,)),
            pltpu.SemaphoreType.DMA((3,)),
            pltpu.SemaphoreType.DMA,
        ],
        compiler_params=pltpu.CompilerParams(use_tc_tiling_on_sc=False),
    )
    return f(n_src, src_r, dst_r, w_r)


def _sage_layer(h, src_r, dst_r, w_r, Qw, Qb, Ww, Wb, add):
    n_src = _mm_relu(h, Qw, Qb)
    acc2, ws2 = _edge_agg(n_src, src_r, dst_r, w_r)
    a = acc2.reshape(2, NP, D)
    w = ws2.reshape(2, NP, WSW)
    return _combine(a[0], a[1], w[0], w[1], h, Ww, Wb, add)


def kernel(nids, edge_index1, weights1, edge_index2, weights2, pos_edges, neg_edges,
           emb_table, bias, Q1w, Q1b, W1w, W1b, Q2w, Q2b, W2w, W2b):
    h0 = jnp.take(emb_table, nids, axis=0)
    zero = jnp.zeros((N, D), jnp.float32)
    esh = (NT * NGRP, SG, CH)
    s1, d1 = edge_index1[0].reshape(esh), edge_index1[1].reshape(esh)
    s2, d2 = edge_index2[0].reshape(esh), edge_index2[1].reshape(esh)
    w1r = weights1.reshape(esh)
    w2r = weights2.reshape(esh)
    h1 = _sage_layer(h0, s1, d1, w1r, Q1w, Q1b, W1w, W1b, zero)
    h_item = _sage_layer(h1, s2, d2, w2r, Q2w, Q2b, W2w, W2b, h0)

    bn = bias[nids, 0]

    def _score(edges):
        s = jnp.sum(h_item[edges[0]] * h_item[edges[1]], axis=1, keepdims=True)
        return s + bn[edges[0], None] + bn[edges[1], None]

    return jnp.concatenate([_score(pos_edges), _score(neg_edges)], axis=0)
